# baseline (speedup 1.0000x reference)
import functools

import jax
import jax.numpy as jnp
from jax import lax
from jax.experimental import pallas as pl
from jax.experimental.pallas import tpu as pltpu
from jax.experimental.pallas import tpu_sc as plsc

_N = 10000
_E = 320000
_D = 128

_NC = 2
_NS = 16
_NW = _NC * _NS
_L = 16

_EPW = _E // _NW
_C = 80
_SC_E = 2000
_NSUPER = _EPW // _SC_E
_NCHUNK = _SC_E // _C
_EB = 2 * _SC_E
_ZR = 80
_WB_STRIDE = 624
_WB_ROWS = 640


def _sc_aggregate(x, edata, ew):
  mesh = plsc.VectorSubcoreMesh(core_axis_name="c", subcore_axis_name="s")

  @functools.partial(
      pl.kernel,
      out_type=[
          jax.ShapeDtypeStruct((_NC, _N, _D), jnp.float32),
          jax.ShapeDtypeStruct((_NC, _N), jnp.float32),
      ],
      mesh=mesh,
      scratch_types=[
          pltpu.VMEM_SHARED((_N, _D), jnp.float32),
          pltpu.VMEM_SHARED((_N,), jnp.float32),
          pltpu.VMEM((_EB,), jnp.int32),
          pltpu.VMEM((_SC_E,), jnp.float32),
          pltpu.VMEM((_C, _D), jnp.float32),
          pltpu.VMEM((_C, _D), jnp.float32),
          pltpu.VMEM((_C, _D), jnp.float32),
          pltpu.VMEM((_C,), jnp.int32),
          pltpu.VMEM((_C,), jnp.int32),
          pltpu.VMEM((_C,), jnp.int32),
          pltpu.VMEM((_ZR, _D), jnp.float32),
          pltpu.VMEM((_WB_ROWS,), jnp.float32),
          pltpu.VMEM((_C,), jnp.float32),
          pltpu.SemaphoreType.DMA,
          pltpu.SemaphoreType.DMA,
          pltpu.SemaphoreType.DMA,
          pltpu.SemaphoreType.DMA,
          pltpu.SemaphoreType.DMA,
          pltpu.SemaphoreType.DMA,
          pltpu.SemaphoreType.DMA,
          pltpu.SemaphoreType.DMA,
          pltpu.SemaphoreType.DMA,
      ],
  )
  def agg(x_hbm, edata_hbm, ew_hbm, acc_out, deg_out,
          acc_sh, deg_sh, ebuf, ewbuf, rows0, rows1, rows2,
          d80_0, d80_1, d80_2, zrows_v, z1d_v, ones_v,
          sg0, sg1, sg2, ss0, ss1, ss2, sd0, sd1, sd2):
    cid = lax.axis_index("c")
    sid = lax.axis_index("s")
    wid = sid * _NC + cid

    rows = (rows0, rows1, rows2)
    d80 = (d80_0, d80_1, d80_2)
    sg = (sg0, sg1, sg2)
    ss = (ss0, ss1, ss2)
    sd = (sd0, sd1, sd2)

    zv = jnp.zeros((_L,), jnp.float32)
    onev = jnp.ones((_L,), jnp.float32)

    def zzero(i, _):
      for j in range(_D // _L):
        zrows_v[i, pl.ds(j * _L, _L)] = zv
      return 0
    lax.fori_loop(0, _ZR, zzero, 0)

    for g in range(_C // _L):
      ones_v[pl.ds(g * _L, _L)] = onev
    for g in range(_WB_ROWS // _L):
      z1d_v[pl.ds(g * _L, _L)] = zv

    for k in range(_WB_ROWS // _ZR):
      off = sid * _WB_STRIDE + k * _ZR
      pltpu.sync_copy(zrows_v, acc_sh.at[pl.ds(off, _ZR)])
    pltpu.sync_copy(z1d_v, deg_sh.at[pl.ds(sid * _WB_STRIDE, _WB_ROWS)])

    plsc.subcore_barrier()

    def issue_gather(w, b):
      pltpu.async_copy(
          x_hbm.at[ebuf.at[pl.ds(w * _C, _C)]], rows[b], sg[b])

    def chunk_op(w, b):
      base = w * _C
      rows_b = rows[b]
      d80_b = d80[b]
      prev = (b + 2) % 3

      pltpu.make_async_copy(
          x_hbm.at[ebuf.at[pl.ds(0, _C)]], rows_b, sg[b]).wait()

      for g in range(_C // _L):
        goff = base + g * _L
        wv16 = jnp.abs(ewbuf[pl.ds(goff, _L)])
        d80_b[pl.ds(g * _L, _L)] = ebuf[pl.ds(_SC_E + goff, _L)]
        for l in range(_L):
          e = g * _L + l
          wv = jnp.full((_L,), wv16[l], jnp.float32)
          for j in range(_D // _L):
            sl = pl.ds(j * _L, _L)
            rows_b[e, sl] = rows_b[e, sl] * wv

      pltpu.async_copy(rows_b, acc_sh.at[d80_b], ss[b], add=True)
      pltpu.async_copy(ones_v, deg_sh.at[d80_b], sd[b], add=True)

      @pl.when(w >= 1)
      def _():
        pltpu.make_async_copy(rows[prev], acc_sh.at[d80[prev]],
                              ss[prev]).wait()
        pltpu.make_async_copy(ones_v, deg_sh.at[d80[prev]],
                              sd[prev]).wait()

      @pl.when(w < _NCHUNK - 2)
      def _():
        issue_gather(w + 2, prev)

    def super_body(s, _):
      blk = wid * _NSUPER + s
      pltpu.sync_copy(edata_hbm.at[pl.ds(blk * _EB, _EB)], ebuf)
      pltpu.sync_copy(ew_hbm.at[pl.ds(blk * _SC_E, _SC_E)], ewbuf)
      issue_gather(0, 0)
      issue_gather(1, 1)

      def inner(w, _):
        m = lax.rem(w, 3)
        for b in range(3):
          @pl.when(m == b)
          def _():
            chunk_op(w, b)
        return 0
      lax.fori_loop(0, _NCHUNK, inner, 0)

      lastb = (_NCHUNK - 1) % 3
      pltpu.make_async_copy(rows[lastb], acc_sh.at[d80[lastb]],
                            ss[lastb]).wait()
      pltpu.make_async_copy(ones_v, deg_sh.at[d80[lastb]],
                            sd[lastb]).wait()
      return 0
    lax.fori_loop(0, _NSUPER, super_body, 0)

    plsc.subcore_barrier()

    off = sid * _WB_STRIDE
    pltpu.sync_copy(acc_sh.at[pl.ds(off, _WB_ROWS)],
                    acc_out.at[cid, pl.ds(off, _WB_ROWS)])

    @pl.when(sid == 0)
    def _():
      pltpu.sync_copy(deg_sh, deg_out.at[cid])

  return agg(x, edata, ew)


_BR = 1000


def _tc_body(acc_ref, deg_ref, w_ref, b_ref, o_ref):
  s = acc_ref[0] + acc_ref[1]
  d = deg_ref[0] + deg_ref[1]
  y = s / jnp.maximum(d, 1.0)
  o_ref[...] = lax.dot_general(
      y, w_ref[...], (((1,), (1,)), ((), ())),
      preferred_element_type=jnp.float32) + b_ref[...]


def _tc_finish(acc, deg, W, b2):
  return pl.pallas_call(
      _tc_body,
      out_shape=jax.ShapeDtypeStruct((_N, _D), jnp.float32),
      grid=(_N // _BR,),
      in_specs=[
          pl.BlockSpec((_NC, _BR, _D), lambda i: (0, i, 0)),
          pl.BlockSpec((_NC, _BR, 1), lambda i: (0, i, 0)),
          pl.BlockSpec((_D, _D), lambda i: (0, 0)),
          pl.BlockSpec((1, _D), lambda i: (0, 0)),
      ],
      out_specs=pl.BlockSpec((_BR, _D), lambda i: (i, 0)),
  )(acc, deg, W, b2)


@jax.jit
def kernel(x, edge_index, edge_weight, W, b):
  srcr = edge_index[1].reshape(_NW, _NSUPER, 1, _SC_E)
  dstr = edge_index[0].reshape(_NW, _NSUPER, 1, _SC_E)
  edata = jnp.concatenate([srcr, dstr], axis=2).reshape(-1)
  acc, deg = _sc_aggregate(x, edata, edge_weight)
  return _tc_finish(acc, deg.reshape(_NC, _N, 1), W, b.reshape(1, _D))


# speedup vs baseline: 1.2877x; 1.2877x over previous
import functools

import jax
import jax.numpy as jnp
from jax import lax
from jax.experimental import pallas as pl
from jax.experimental.pallas import tpu as pltpu
from jax.experimental.pallas import tpu_sc as plsc

_N = 10000
_E = 320000
_D = 128

_NC = 2
_NS = 16
_NW = _NC * _NS
_L = 16

_EPW = _E // _NW
_C = 80
_SC_E = 2000
_NSUPER = _EPW // _SC_E
_NCHUNK = _SC_E // _C
_EB = 2 * _SC_E
_ZR = 80
_WB_STRIDE = 624
_WB_ROWS = 640


def _sc_aggregate(x, edata, ew):
  mesh = plsc.VectorSubcoreMesh(core_axis_name="c", subcore_axis_name="s")

  @functools.partial(
      pl.kernel,
      out_type=[
          jax.ShapeDtypeStruct((_NC, _N, _D), jnp.float32),
          jax.ShapeDtypeStruct((_NC, _N), jnp.float32),
      ],
      mesh=mesh,
      scratch_types=[
          pltpu.VMEM_SHARED((_N, _D), jnp.float32),
          pltpu.VMEM_SHARED((_N,), jnp.float32),
          pltpu.VMEM((_EB,), jnp.int32),
          pltpu.VMEM((_SC_E,), jnp.float32),
          pltpu.VMEM((_C, _D), jnp.float32),
          pltpu.VMEM((_C, _D), jnp.float32),
          pltpu.VMEM((_C, _D), jnp.float32),
          pltpu.VMEM((_C,), jnp.int32),
          pltpu.VMEM((_C,), jnp.int32),
          pltpu.VMEM((_C,), jnp.int32),
          pltpu.VMEM((_ZR, _D), jnp.float32),
          pltpu.VMEM((_WB_ROWS,), jnp.float32),
          pltpu.VMEM((_C,), jnp.float32),
          pltpu.SemaphoreType.DMA,
          pltpu.SemaphoreType.DMA,
          pltpu.SemaphoreType.DMA,
          pltpu.SemaphoreType.DMA,
          pltpu.SemaphoreType.DMA,
          pltpu.SemaphoreType.DMA,
          pltpu.SemaphoreType.DMA,
          pltpu.SemaphoreType.DMA,
          pltpu.SemaphoreType.DMA,
      ],
  )
  def agg(x_hbm, edata_hbm, ew_hbm, acc_out, deg_out,
          acc_sh, deg_sh, ebuf, ewbuf, rows0, rows1, rows2,
          d80_0, d80_1, d80_2, zrows_v, z1d_v, ones_v,
          sg0, sg1, sg2, ss0, ss1, ss2, sd0, sd1, sd2):
    cid = lax.axis_index("c")
    sid = lax.axis_index("s")
    wid = sid * _NC + cid

    rows = (rows0, rows1, rows2)
    d80 = (d80_0, d80_1, d80_2)
    sg = (sg0, sg1, sg2)
    ss = (ss0, ss1, ss2)
    sd = (sd0, sd1, sd2)

    zv = jnp.zeros((_L,), jnp.float32)
    onev = jnp.ones((_L,), jnp.float32)

    def zzero(i, _):
      for j in range(_D // _L):
        zrows_v[i, pl.ds(j * _L, _L)] = zv
      return 0
    lax.fori_loop(0, _ZR, zzero, 0)

    for g in range(_C // _L):
      ones_v[pl.ds(g * _L, _L)] = onev
    for g in range(_WB_ROWS // _L):
      z1d_v[pl.ds(g * _L, _L)] = zv

    for k in range(_WB_ROWS // _ZR):
      off = sid * _WB_STRIDE + k * _ZR
      pltpu.sync_copy(zrows_v, acc_sh.at[pl.ds(off, _ZR)])
    pltpu.sync_copy(z1d_v, deg_sh.at[pl.ds(sid * _WB_STRIDE, _WB_ROWS)])

    plsc.subcore_barrier()

    def issue_gather(w, b):
      pltpu.async_copy(
          x_hbm.at[ebuf.at[pl.ds(w * _C, _C)]], rows[b], sg[b])

    def chunk_op(w, b):
      base = w * _C
      rows_b = rows[b]
      d80_b = d80[b]
      prev = (b + 2) % 3

      pltpu.make_async_copy(
          x_hbm.at[ebuf.at[pl.ds(0, _C)]], rows_b, sg[b]).wait()

      def grp_body(g, _):
        goff = base + g * _L
        wv16 = jnp.abs(ewbuf[pl.ds(goff, _L)])
        d80_b[pl.ds(g * _L, _L)] = ebuf[pl.ds(_SC_E + goff, _L)]
        for l in range(_L):
          e = g * _L + l
          wv = jnp.full((_L,), wv16[l], jnp.float32)
          for j in range(_D // _L):
            sl = pl.ds(j * _L, _L)
            rows_b[e, sl] = rows_b[e, sl] * wv
        return 0
      lax.fori_loop(0, _C // _L, grp_body, 0)

      pltpu.async_copy(rows_b, acc_sh.at[d80_b], ss[b], add=True)
      pltpu.async_copy(ones_v, deg_sh.at[d80_b], sd[b], add=True)

      @pl.when(w >= 1)
      def _():
        pltpu.make_async_copy(rows[prev], acc_sh.at[d80[prev]],
                              ss[prev]).wait()
        pltpu.make_async_copy(ones_v, deg_sh.at[d80[prev]],
                              sd[prev]).wait()

      @pl.when(w < _NCHUNK - 2)
      def _():
        issue_gather(w + 2, prev)

    def super_body(s, _):
      blk = wid * _NSUPER + s
      pltpu.sync_copy(edata_hbm.at[pl.ds(blk * _EB, _EB)], ebuf)
      pltpu.sync_copy(ew_hbm.at[pl.ds(blk * _SC_E, _SC_E)], ewbuf)
      issue_gather(0, 0)
      issue_gather(1, 1)

      def inner(w, _):
        m = lax.rem(w, 3)
        for b in range(3):
          @pl.when(m == b)
          def _():
            chunk_op(w, b)
        return 0
      lax.fori_loop(0, _NCHUNK, inner, 0)

      lastb = (_NCHUNK - 1) % 3
      pltpu.make_async_copy(rows[lastb], acc_sh.at[d80[lastb]],
                            ss[lastb]).wait()
      pltpu.make_async_copy(ones_v, deg_sh.at[d80[lastb]],
                            sd[lastb]).wait()
      return 0
    lax.fori_loop(0, _NSUPER, super_body, 0)

    plsc.subcore_barrier()

    off = sid * _WB_STRIDE
    pltpu.sync_copy(acc_sh.at[pl.ds(off, _WB_ROWS)],
                    acc_out.at[cid, pl.ds(off, _WB_ROWS)])

    @pl.when(sid == 0)
    def _():
      pltpu.sync_copy(deg_sh, deg_out.at[cid])

  return agg(x, edata, ew)


_BR = 1000


def _tc_body(acc_ref, deg_ref, w_ref, b_ref, o_ref):
  s = acc_ref[0] + acc_ref[1]
  d = deg_ref[0] + deg_ref[1]
  y = s / jnp.maximum(d, 1.0)
  o_ref[...] = lax.dot_general(
      y, w_ref[...], (((1,), (1,)), ((), ())),
      preferred_element_type=jnp.float32) + b_ref[...]


def _tc_finish(acc, deg, W, b2):
  return pl.pallas_call(
      _tc_body,
      out_shape=jax.ShapeDtypeStruct((_N, _D), jnp.float32),
      grid=(_N // _BR,),
      in_specs=[
          pl.BlockSpec((_NC, _BR, _D), lambda i: (0, i, 0)),
          pl.BlockSpec((_NC, _BR, 1), lambda i: (0, i, 0)),
          pl.BlockSpec((_D, _D), lambda i: (0, 0)),
          pl.BlockSpec((1, _D), lambda i: (0, 0)),
      ],
      out_specs=pl.BlockSpec((_BR, _D), lambda i: (i, 0)),
  )(acc, deg, W, b2)


@jax.jit
def kernel(x, edge_index, edge_weight, W, b):
  srcr = edge_index[1].reshape(_NW, _NSUPER, 1, _SC_E)
  dstr = edge_index[0].reshape(_NW, _NSUPER, 1, _SC_E)
  edata = jnp.concatenate([srcr, dstr], axis=2).reshape(-1)
  acc, deg = _sc_aggregate(x, edata, edge_weight)
  return _tc_finish(acc, deg.reshape(_NC, _N, 1), W, b.reshape(1, _D))


# speedup vs baseline: 1.3346x; 1.0365x over previous
import functools

import jax
import jax.numpy as jnp
from jax import lax
from jax.experimental import pallas as pl
from jax.experimental.pallas import tpu as pltpu
from jax.experimental.pallas import tpu_sc as plsc

_N = 10000
_E = 320000
_D = 128

_NC = 2
_NS = 16
_NW = _NC * _NS
_L = 16

_EPW = _E // _NW
_C = 80
_SC_E = 2000
_NSUPER = _EPW // _SC_E
_NCHUNK = _SC_E // _C
_EB = 2 * _SC_E
_ZR = 80
_WB_STRIDE = 624
_WB_ROWS = 640


def _sc_aggregate(x, edata, ew):
  mesh = plsc.VectorSubcoreMesh(core_axis_name="c", subcore_axis_name="s")

  @functools.partial(
      pl.kernel,
      out_type=[
          jax.ShapeDtypeStruct((_NC, _N, _D), jnp.float32),
          jax.ShapeDtypeStruct((_NC, _N), jnp.float32),
      ],
      mesh=mesh,
      scratch_types=[
          pltpu.VMEM_SHARED((_N, _D), jnp.float32),
          pltpu.VMEM_SHARED((_N,), jnp.float32),
          pltpu.VMEM((2 * _EB,), jnp.int32),
          pltpu.VMEM((2 * _SC_E,), jnp.float32),
          pltpu.VMEM((_C, _D), jnp.float32),
          pltpu.VMEM((_C, _D), jnp.float32),
          pltpu.VMEM((_C, _D), jnp.float32),
          pltpu.VMEM((_C,), jnp.int32),
          pltpu.VMEM((_C,), jnp.int32),
          pltpu.VMEM((_C,), jnp.int32),
          pltpu.VMEM((_WB_ROWS,), jnp.float32),
          pltpu.VMEM((_C,), jnp.float32),
          pltpu.SemaphoreType.DMA,
          pltpu.SemaphoreType.DMA,
          pltpu.SemaphoreType.DMA,
          pltpu.SemaphoreType.DMA,
          pltpu.SemaphoreType.DMA,
          pltpu.SemaphoreType.DMA,
          pltpu.SemaphoreType.DMA,
          pltpu.SemaphoreType.DMA,
          pltpu.SemaphoreType.DMA,
          pltpu.SemaphoreType.DMA,
          pltpu.SemaphoreType.DMA,
      ],
  )
  def agg(x_hbm, edata_hbm, ew_hbm, acc_out, deg_out,
          acc_sh, deg_sh, ebuf, ewbuf, rows0, rows1, rows2,
          d80_0, d80_1, d80_2, z1d_v, ones_v,
          sg0, sg1, sg2, ss0, ss1, ss2, sd0, sd1, sd2, sea, seb):
    cid = lax.axis_index("c")
    sid = lax.axis_index("s")
    wid = sid * _NC + cid

    rows = (rows0, rows1, rows2)
    d80 = (d80_0, d80_1, d80_2)
    sg = (sg0, sg1, sg2)
    ss = (ss0, ss1, ss2)
    sd = (sd0, sd1, sd2)

    zv = jnp.zeros((_L,), jnp.float32)
    onev = jnp.ones((_L,), jnp.float32)

    def stage(s):
      slot = lax.rem(s, 2)
      blk = wid * _NSUPER + s
      pltpu.async_copy(edata_hbm.at[pl.ds(blk * _EB, _EB)],
                       ebuf.at[pl.ds(slot * _EB, _EB)], sea)
      pltpu.async_copy(ew_hbm.at[pl.ds(blk * _SC_E, _SC_E)],
                       ewbuf.at[pl.ds(slot * _SC_E, _SC_E)], seb)

    def stage_wait(s):
      slot = lax.rem(s, 2)
      blk = wid * _NSUPER + s
      pltpu.make_async_copy(edata_hbm.at[pl.ds(blk * _EB, _EB)],
                            ebuf.at[pl.ds(slot * _EB, _EB)], sea).wait()
      pltpu.make_async_copy(ew_hbm.at[pl.ds(blk * _SC_E, _SC_E)],
                            ewbuf.at[pl.ds(slot * _SC_E, _SC_E)], seb).wait()

    stage(0)

    def zzero(i, _):
      for j in range(_D // _L):
        rows0[i, pl.ds(j * _L, _L)] = zv
      return 0
    lax.fori_loop(0, _ZR, zzero, 0)

    for g in range(_C // _L):
      ones_v[pl.ds(g * _L, _L)] = onev
    for g in range(_WB_ROWS // _L):
      z1d_v[pl.ds(g * _L, _L)] = zv

    for k in range(_WB_ROWS // _ZR):
      off = sid * _WB_STRIDE + k * _ZR
      pltpu.sync_copy(rows0, acc_sh.at[pl.ds(off, _ZR)])
    pltpu.sync_copy(z1d_v, deg_sh.at[pl.ds(sid * _WB_STRIDE, _WB_ROWS)])

    plsc.subcore_barrier()

    def issue_gather(eoff, w, b):
      pltpu.async_copy(
          x_hbm.at[ebuf.at[pl.ds(eoff + w * _C, _C)]], rows[b], sg[b])

    def chunk_op(eoff, woff, w, b):
      base = w * _C
      rows_b = rows[b]
      d80_b = d80[b]
      prev = (b + 2) % 3

      pltpu.make_async_copy(
          x_hbm.at[ebuf.at[pl.ds(0, _C)]], rows_b, sg[b]).wait()

      def grp_body(g, _):
        goff = base + g * _L
        wv16 = jnp.abs(ewbuf[pl.ds(woff + goff, _L)])
        d80_b[pl.ds(g * _L, _L)] = ebuf[pl.ds(eoff + _SC_E + goff, _L)]
        for l in range(_L):
          e = g * _L + l
          wv = jnp.full((_L,), wv16[l], jnp.float32)
          for j in range(_D // _L):
            sl = pl.ds(j * _L, _L)
            rows_b[e, sl] = rows_b[e, sl] * wv
        return 0
      lax.fori_loop(0, _C // _L, grp_body, 0)

      pltpu.async_copy(rows_b, acc_sh.at[d80_b], ss[b], add=True)
      pltpu.async_copy(ones_v, deg_sh.at[d80_b], sd[b], add=True)

      @pl.when(w >= 1)
      def _():
        pltpu.make_async_copy(rows[prev], acc_sh.at[d80[prev]],
                              ss[prev]).wait()
        pltpu.make_async_copy(ones_v, deg_sh.at[d80[prev]],
                              sd[prev]).wait()

      @pl.when(w < _NCHUNK - 2)
      def _():
        issue_gather(eoff, w + 2, prev)

    def super_body(s, _):
      eoff = lax.rem(s, 2) * _EB
      woff = lax.rem(s, 2) * _SC_E
      stage_wait(s)
      @pl.when(s < _NSUPER - 1)
      def _():
        stage(s + 1)
      issue_gather(eoff, 0, 0)
      issue_gather(eoff, 1, 1)

      def inner(w, _):
        m = lax.rem(w, 3)
        for b in range(3):
          @pl.when(m == b)
          def _():
            chunk_op(eoff, woff, w, b)
        return 0
      lax.fori_loop(0, _NCHUNK, inner, 0)

      lastb = (_NCHUNK - 1) % 3
      pltpu.make_async_copy(rows[lastb], acc_sh.at[d80[lastb]],
                            ss[lastb]).wait()
      pltpu.make_async_copy(ones_v, deg_sh.at[d80[lastb]],
                            sd[lastb]).wait()
      return 0
    lax.fori_loop(0, _NSUPER, super_body, 0)

    plsc.subcore_barrier()

    off = sid * _WB_STRIDE
    pltpu.sync_copy(acc_sh.at[pl.ds(off, _WB_ROWS)],
                    acc_out.at[cid, pl.ds(off, _WB_ROWS)])

    @pl.when(sid == 0)
    def _():
      pltpu.sync_copy(deg_sh, deg_out.at[cid])

  return agg(x, edata, ew)


_BR = 1000


def _tc_body(acc_ref, deg_ref, w_ref, b_ref, o_ref):
  s = acc_ref[0] + acc_ref[1]
  d = deg_ref[0] + deg_ref[1]
  y = s / jnp.maximum(d, 1.0)
  o_ref[...] = lax.dot_general(
      y, w_ref[...], (((1,), (1,)), ((), ())),
      preferred_element_type=jnp.float32) + b_ref[...]


def _tc_finish(acc, deg, W, b2):
  return pl.pallas_call(
      _tc_body,
      out_shape=jax.ShapeDtypeStruct((_N, _D), jnp.float32),
      grid=(_N // _BR,),
      in_specs=[
          pl.BlockSpec((_NC, _BR, _D), lambda i: (0, i, 0)),
          pl.BlockSpec((_NC, _BR, 1), lambda i: (0, i, 0)),
          pl.BlockSpec((_D, _D), lambda i: (0, 0)),
          pl.BlockSpec((1, _D), lambda i: (0, 0)),
      ],
      out_specs=pl.BlockSpec((_BR, _D), lambda i: (i, 0)),
  )(acc, deg, W, b2)


@jax.jit
def kernel(x, edge_index, edge_weight, W, b):
  srcr = edge_index[1].reshape(_NW, _NSUPER, 1, _SC_E)
  dstr = edge_index[0].reshape(_NW, _NSUPER, 1, _SC_E)
  edata = jnp.concatenate([srcr, dstr], axis=2).reshape(-1)
  acc, deg = _sc_aggregate(x, edata, edge_weight)
  return _tc_finish(acc, deg.reshape(_NC, _N, 1), W, b.reshape(1, _D))


# speedup vs baseline: 1.3686x; 1.0254x over previous
import functools

import jax
import jax.numpy as jnp
from jax import lax
from jax.experimental import pallas as pl
from jax.experimental.pallas import tpu as pltpu
from jax.experimental.pallas import tpu_sc as plsc

_N = 10000
_E = 320000
_D = 128

_NC = 2
_NS = 16
_NW = _NC * _NS
_L = 16

_EPW = _E // _NW
_C = 80
_SC_E = 2000
_NSUPER = _EPW // _SC_E
_NCHUNK = _SC_E // _C
_EB = 2 * _SC_E
_ZR = 80
_WB_STRIDE = 624
_WB_ROWS = 640


def _sc_aggregate(x, edata, ew):
  mesh = plsc.VectorSubcoreMesh(core_axis_name="c", subcore_axis_name="s")

  @functools.partial(
      pl.kernel,
      out_type=[
          jax.ShapeDtypeStruct((_NC, _N, _D), jnp.float32),
          jax.ShapeDtypeStruct((_NC, _N), jnp.float32),
      ],
      mesh=mesh,
      scratch_types=[
          pltpu.VMEM_SHARED((_N, _D), jnp.float32),
          pltpu.VMEM_SHARED((_N,), jnp.float32),
          pltpu.VMEM((2 * _EB,), jnp.int32),
          pltpu.VMEM((2 * _SC_E,), jnp.float32),
          pltpu.VMEM((_C, _D), jnp.float32),
          pltpu.VMEM((_C, _D), jnp.float32),
          pltpu.VMEM((_C, _D), jnp.float32),
          pltpu.VMEM((_C,), jnp.int32),
          pltpu.VMEM((_C,), jnp.int32),
          pltpu.VMEM((_C,), jnp.int32),
          pltpu.VMEM((_WB_ROWS,), jnp.float32),
          pltpu.VMEM((_C,), jnp.float32),
          pltpu.SemaphoreType.DMA,
          pltpu.SemaphoreType.DMA,
          pltpu.SemaphoreType.DMA,
          pltpu.SemaphoreType.DMA,
          pltpu.SemaphoreType.DMA,
          pltpu.SemaphoreType.DMA,
          pltpu.SemaphoreType.DMA,
          pltpu.SemaphoreType.DMA,
          pltpu.SemaphoreType.DMA,
          pltpu.SemaphoreType.DMA,
          pltpu.SemaphoreType.DMA,
      ],
  )
  def agg(x_hbm, edata_hbm, ew_hbm, acc_out, deg_out,
          acc_sh, deg_sh, ebuf, ewbuf, rows0, rows1, rows2,
          d80_0, d80_1, d80_2, z1d_v, ones_v,
          sg0, sg1, sg2, ss0, ss1, ss2, sd0, sd1, sd2, sea, seb):
    cid = lax.axis_index("c")
    sid = lax.axis_index("s")
    wid = sid * _NC + cid

    rows = (rows0, rows1, rows2)
    d80 = (d80_0, d80_1, d80_2)
    sg = (sg0, sg1, sg2)
    ss = (ss0, ss1, ss2)
    sd = (sd0, sd1, sd2)

    zv = jnp.zeros((_L,), jnp.float32)
    onev = jnp.ones((_L,), jnp.float32)

    def stage(s):
      slot = lax.rem(s, 2)
      blk = wid * _NSUPER + s
      pltpu.async_copy(edata_hbm.at[pl.ds(blk * _EB, _EB)],
                       ebuf.at[pl.ds(slot * _EB, _EB)], sea)
      pltpu.async_copy(ew_hbm.at[pl.ds(blk * _SC_E, _SC_E)],
                       ewbuf.at[pl.ds(slot * _SC_E, _SC_E)], seb)

    def stage_wait(s):
      slot = lax.rem(s, 2)
      blk = wid * _NSUPER + s
      pltpu.make_async_copy(edata_hbm.at[pl.ds(blk * _EB, _EB)],
                            ebuf.at[pl.ds(slot * _EB, _EB)], sea).wait()
      pltpu.make_async_copy(ew_hbm.at[pl.ds(blk * _SC_E, _SC_E)],
                            ewbuf.at[pl.ds(slot * _SC_E, _SC_E)], seb).wait()

    stage(0)

    def zzero(i, _):
      for j in range(_D // _L):
        rows0[i, pl.ds(j * _L, _L)] = zv
      return 0
    lax.fori_loop(0, _ZR, zzero, 0)

    for g in range(_C // _L):
      ones_v[pl.ds(g * _L, _L)] = onev
    for g in range(_WB_ROWS // _L):
      z1d_v[pl.ds(g * _L, _L)] = zv

    for k in range(_WB_ROWS // _ZR):
      off = sid * _WB_STRIDE + k * _ZR
      pltpu.sync_copy(rows0, acc_sh.at[pl.ds(off, _ZR)])
    pltpu.sync_copy(z1d_v, deg_sh.at[pl.ds(sid * _WB_STRIDE, _WB_ROWS)])

    plsc.subcore_barrier()

    def issue_gather(eoff, w, b):
      pltpu.async_copy(
          x_hbm.at[ebuf.at[pl.ds(eoff + w * _C, _C)]], rows[b], sg[b])

    _TOT = _NSUPER * _NCHUNK

    def chunk_op(wg, b):
      s = lax.div(wg, _NCHUNK)
      slot = lax.rem(s, 2)
      eoff = slot * _EB
      woff = slot * _SC_E
      base = (wg - s * _NCHUNK) * _C
      rows_b = rows[b]
      d80_b = d80[b]
      prev = (b + 2) % 3

      @pl.when(jnp.logical_and(base == 0, s >= 1))
      def _():
        @pl.when(s < _NSUPER - 1)
        def _():
          stage(s + 1)

      pltpu.make_async_copy(
          x_hbm.at[ebuf.at[pl.ds(0, _C)]], rows_b, sg[b]).wait()

      def grp_body(g, _):
        goff = base + g * _L
        wv16 = jnp.abs(ewbuf[pl.ds(woff + goff, _L)])
        d80_b[pl.ds(g * _L, _L)] = ebuf[pl.ds(eoff + _SC_E + goff, _L)]
        for l in range(_L):
          e = g * _L + l
          wv = jnp.full((_L,), wv16[l], jnp.float32)
          for j in range(_D // _L):
            sl = pl.ds(j * _L, _L)
            rows_b[e, sl] = rows_b[e, sl] * wv
        return 0
      lax.fori_loop(0, _C // _L, grp_body, 0)

      pltpu.async_copy(rows_b, acc_sh.at[d80_b], ss[b], add=True)
      pltpu.async_copy(ones_v, deg_sh.at[d80_b], sd[b], add=True)

      @pl.when(wg >= 1)
      def _():
        pltpu.make_async_copy(rows[prev], acc_sh.at[d80[prev]],
                              ss[prev]).wait()
        pltpu.make_async_copy(ones_v, deg_sh.at[d80[prev]],
                              sd[prev]).wait()

      @pl.when(wg < _TOT - 2)
      def _():
        wg2 = wg + 2
        s2 = lax.div(wg2, _NCHUNK)
        wl2 = wg2 - s2 * _NCHUNK
        @pl.when(wl2 == 0)
        def _():
          stage_wait(s2)
        issue_gather(lax.rem(s2, 2) * _EB, wl2, prev)

    stage_wait(0)
    stage(1)
    issue_gather(0, 0, 0)
    issue_gather(0, 1, 1)

    def inner(wg, _):
      m = lax.rem(wg, 3)
      for b in range(3):
        @pl.when(m == b)
        def _():
          chunk_op(wg, b)
      return 0
    lax.fori_loop(0, _TOT, inner, 0)

    lastb = (_TOT - 1) % 3
    pltpu.make_async_copy(rows[lastb], acc_sh.at[d80[lastb]],
                          ss[lastb]).wait()
    pltpu.make_async_copy(ones_v, deg_sh.at[d80[lastb]],
                          sd[lastb]).wait()

    plsc.subcore_barrier()

    off = sid * _WB_STRIDE
    pltpu.sync_copy(acc_sh.at[pl.ds(off, _WB_ROWS)],
                    acc_out.at[cid, pl.ds(off, _WB_ROWS)])

    @pl.when(sid == 0)
    def _():
      pltpu.sync_copy(deg_sh, deg_out.at[cid])

  return agg(x, edata, ew)


_BR = 1000


def _tc_body(acc_ref, deg_ref, w_ref, b_ref, o_ref):
  s = acc_ref[0] + acc_ref[1]
  d = deg_ref[0] + deg_ref[1]
  y = s / jnp.maximum(d, 1.0)
  o_ref[...] = lax.dot_general(
      y, w_ref[...], (((1,), (1,)), ((), ())),
      preferred_element_type=jnp.float32) + b_ref[...]


def _tc_finish(acc, deg, W, b2):
  return pl.pallas_call(
      _tc_body,
      out_shape=jax.ShapeDtypeStruct((_N, _D), jnp.float32),
      grid=(_N // _BR,),
      in_specs=[
          pl.BlockSpec((_NC, _BR, _D), lambda i: (0, i, 0)),
          pl.BlockSpec((_NC, _BR, 1), lambda i: (0, i, 0)),
          pl.BlockSpec((_D, _D), lambda i: (0, 0)),
          pl.BlockSpec((1, _D), lambda i: (0, 0)),
      ],
      out_specs=pl.BlockSpec((_BR, _D), lambda i: (i, 0)),
  )(acc, deg, W, b2)


@jax.jit
def kernel(x, edge_index, edge_weight, W, b):
  srcr = edge_index[1].reshape(_NW, _NSUPER, 1, _SC_E)
  dstr = edge_index[0].reshape(_NW, _NSUPER, 1, _SC_E)
  edata = jnp.concatenate([srcr, dstr], axis=2).reshape(-1)
  acc, deg = _sc_aggregate(x, edata, edge_weight)
  return _tc_finish(acc, deg.reshape(_NC, _N, 1), W, b.reshape(1, _D))
